# baseline (speedup 1.0000x reference)
import jax
import jax.numpy as jnp
import numpy as np
from jax import lax
from jax.experimental import pallas as pl
from jax.experimental.pallas import tpu as pltpu
from jax.experimental.pallas import tpu_sc as plsc

_NC = 2
_NS = 16
_CHUNK = 64
_SCH = 8
_NBUF = 4
_LANES = 16


def _make_sc_agg(n_rows_pad, cw, real_chunks, dst_base, feat, with_deg):
    mesh = plsc.VectorSubcoreMesh(core_axis_name="c", subcore_axis_name="s")
    outs = [jax.ShapeDtypeStruct((_NC, n_rows_pad, feat), jnp.float32)]
    if with_deg:
        outs.append(jax.ShapeDtypeStruct((_NC, _NS, n_rows_pad), jnp.float32))
    scratch = (
        [pltpu.VMEM((_SCH, _CHUNK), jnp.int32) for _ in range(4)]
        + [pltpu.VMEM((_CHUNK, feat), jnp.float32) for _ in range(_NBUF)]
        + [pltpu.VMEM_SHARED((n_rows_pad, feat), jnp.float32)]
        + [pltpu.SemaphoreType.DMA for _ in range(2 * _NBUF + 2)]
    )
    if with_deg:
        scratch.append(pltpu.VMEM((n_rows_pad,), jnp.float32))

    nsc = cw // _SCH
    stripe = n_rows_pad // _NS
    blocks = []
    r = 0
    while r < stripe:
        b = min(_CHUNK, stripe - r)
        blocks.append((r, b))
        r += b

    def body(tab_hbm, idx_hbm, psrc_hbm, pdst_hbm, *refs):
        if with_deg:
            agg_out, deg_out = refs[0], refs[1]
            rs = refs[2:]
            deg_v = rs[4 + _NBUF + 1 + 2 * _NBUF + 2]
        else:
            agg_out = refs[0]
            deg_out = deg_v = None
            rs = refs[1:]
        srcs = [rs[0], rs[1]]
        dsts = [rs[2], rs[3]]
        bufs = list(rs[4:4 + _NBUF])
        agg_s = rs[4 + _NBUF]
        gsems = list(rs[5 + _NBUF:5 + 2 * _NBUF])
        ssems = list(rs[5 + 2 * _NBUF:5 + 3 * _NBUF])
        isems = list(rs[5 + 3 * _NBUF:7 + 3 * _NBUF])

        cid = lax.axis_index("c")
        sid = lax.axis_index("s")
        wid = sid * _NC + cid
        base = sid * stripe

        zv = jnp.zeros((_LANES,), jnp.float32)
        ov = jnp.ones((_LANES,), jnp.float32)

        def _zero_row(i, c):
            for k2 in range(feat // _LANES):
                bufs[0][i, pl.ds(k2 * _LANES, _LANES)] = zv
            return c

        lax.fori_loop(0, _CHUNK, _zero_row, 0)
        for off, b in blocks:
            pltpu.sync_copy(bufs[0].at[pl.ds(0, b)],
                            agg_s.at[pl.ds(base + off, b)])
        if with_deg:
            def _zero_deg(i, c):
                deg_v[pl.ds(i * _LANES, _LANES)] = zv
                return c

            lax.fori_loop(0, n_rows_pad // _LANES, _zero_deg, 0)

        plsc.subcore_barrier()

        off0 = wid * cw

        @pl.when(off0 < real_chunks)
        def _():
            pltpu.sync_copy(idx_hbm.at[pl.ds(off0, _SCH)], srcs[0])
            pltpu.sync_copy(idx_hbm.at[pl.ds(dst_base + off0, _SCH)],
                            dsts[0])

        @pl.when(off0 >= real_chunks)
        def _():
            pltpu.sync_copy(psrc_hbm, srcs[0])
            pltpu.sync_copy(pdst_hbm, dsts[0])
        pltpu.async_copy(tab_hbm.at[srcs[0].at[0]], bufs[0], gsems[0])
        pltpu.async_copy(tab_hbm.at[srcs[0].at[1]], bufs[1], gsems[1])

        def _outer(i, c):
            for s2 in range(2):
                sc_i = 2 * i + s2
                tb = sc_i * _SCH
                sv, dv = srcs[s2], dsts[s2]
                nsv, ndv = srcs[1 - s2], dsts[1 - s2]
                for j in range(_SCH):
                    t = tb + j
                    b = j % _NBUF
                    b2 = (j + 2) % _NBUF
                    pltpu.make_async_copy(
                        tab_hbm.at[sv.at[0]], bufs[b], gsems[b]).wait()
                    pltpu.async_copy(bufs[b], agg_s.at[dv.at[j]],
                                     ssems[b], add=True)
                    if with_deg:
                        for k2 in range(_CHUNK // _LANES):
                            idx16 = dv[j, pl.ds(k2 * _LANES, _LANES)]
                            plsc.addupdate_scatter(deg_v, [idx16], ov)

                    @pl.when(t + 2 < cw)
                    def _():
                        @pl.when(t >= 2)
                        def _():
                            pltpu.make_async_copy(
                                bufs[b2], agg_s.at[dv.at[0]],
                                ssems[b2]).wait()
                        if j < _SCH - 2:
                            pltpu.async_copy(tab_hbm.at[sv.at[j + 2]],
                                             bufs[b2], gsems[b2])
                        else:
                            if j == _SCH - 2:
                                pltpu.make_async_copy(
                                    psrc_hbm, nsv, isems[1 - s2]).wait()
                                pltpu.make_async_copy(
                                    pdst_hbm, ndv, isems[1 - s2]).wait()
                            pltpu.async_copy(
                                tab_hbm.at[nsv.at[j + 2 - _SCH]],
                                bufs[b2], gsems[b2])

                    if j == 1:
                        @pl.when(sc_i + 1 < nsc)
                        def _():
                            offc = wid * cw + (sc_i + 1) * _SCH

                            @pl.when(offc < real_chunks)
                            def _():
                                pltpu.async_copy(
                                    idx_hbm.at[pl.ds(offc, _SCH)],
                                    nsv, isems[1 - s2])
                                pltpu.async_copy(
                                    idx_hbm.at[pl.ds(dst_base + offc, _SCH)],
                                    ndv, isems[1 - s2])

                            @pl.when(offc >= real_chunks)
                            def _():
                                pltpu.async_copy(psrc_hbm, nsv,
                                                 isems[1 - s2])
                                pltpu.async_copy(pdst_hbm, ndv,
                                                 isems[1 - s2])
            return c

        lax.fori_loop(0, nsc // 2, _outer, 0)

        for b in range(_NBUF):
            pltpu.make_async_copy(bufs[b], agg_s.at[dsts[0].at[0]],
                                  ssems[b]).wait()

        plsc.subcore_barrier()

        pltpu.sync_copy(agg_s.at[pl.ds(base, stripe)],
                        agg_out.at[cid, pl.ds(base, stripe)])
        if with_deg:
            pltpu.sync_copy(deg_v, deg_out.at[cid, sid])

    return pl.kernel(body, out_type=tuple(outs) if with_deg else outs[0],
                     mesh=mesh, scratch_types=scratch,
                     compiler_params=pltpu.CompilerParams(
                         needs_layout_passes=False))


def _lrelu(t):
    return jnp.where(t > 0, t, 0.1 * t)


def _mlp(za, zb, w1T, b1, g, be, w2T, b2, w3T, b3):
    d = za.shape[-1]
    t = (jnp.dot(za, w1T[:d], preferred_element_type=jnp.float32)
         + jnp.dot(zb, w1T[d:], preferred_element_type=jnp.float32) + b1)
    mu = jnp.mean(t, axis=-1, keepdims=True)
    var = jnp.mean((t - mu) ** 2, axis=-1, keepdims=True)
    t = (t - mu) / jnp.sqrt(var + 1e-5) * g + be
    t = _lrelu(t)
    u = _lrelu(jnp.dot(t, w2T, preferred_element_type=jnp.float32) + b2)
    return jnp.dot(u, w3T, preferred_element_type=jnp.float32) + b3


def _tc_h0_body(x_ref, inWT_ref, inb_ref, h0_ref):
    h0_ref[...] = jnp.dot(x_ref[...], inWT_ref[...],
                          preferred_element_type=jnp.float32) + inb_ref[...]


def _tc_layer1_body(x_ref, p_ref, degp_ref, inWT_ref, inb_ref,
                    w1T_ref, b1_ref, g_ref, be_ref, w2T_ref, b2_ref,
                    w3T_ref, b3_ref, h1_ref):
    x = x_ref[...]
    d = x.shape[-1]
    aggx = p_ref[0] + p_ref[1]
    deg = jnp.sum(degp_ref[...], axis=(0, 1))[:, None]
    w1T = w1T_ref[...]
    w1m = w1T[:d] + jnp.dot(inWT_ref[...], w1T[d:],
                            preferred_element_type=jnp.float32)
    c1 = jnp.dot(inb_ref[...], w1T[d:], preferred_element_type=jnp.float32)
    za = x + aggx
    t = (jnp.dot(za, w1m, preferred_element_type=jnp.float32)
         + (1.0 + deg) * c1 + b1_ref[...])
    mu = jnp.mean(t, axis=-1, keepdims=True)
    var = jnp.mean((t - mu) ** 2, axis=-1, keepdims=True)
    t = (t - mu) / jnp.sqrt(var + 1e-5) * g_ref[...] + be_ref[...]
    t = _lrelu(t)
    u = _lrelu(jnp.dot(t, w2T_ref[...],
                       preferred_element_type=jnp.float32) + b2_ref[...])
    h1_ref[...] = jnp.dot(u, w3T_ref[...],
                          preferred_element_type=jnp.float32) + b3_ref[...]


def _tc_outpart_body(h0_ref, h1_ref, outWT_ref, outb_ref, op_ref):
    d = h0_ref.shape[-1]
    outWT = outWT_ref[...]
    op_ref[...] = (jnp.dot(h0_ref[...], outWT[:d],
                           preferred_element_type=jnp.float32)
                   + jnp.dot(h1_ref[...], outWT[d:2 * d],
                             preferred_element_type=jnp.float32)
                   + outb_ref[...])


def _tc_layer2_body(x_ref, h1_ref, p_ref, q_ref, op_ref,
                    w1T_ref, b1_ref, g_ref, be_ref, w2T_ref, b2_ref,
                    w3T_ref, b3_ref, outWT_ref, o_ref):
    x = x_ref[...]
    d = x.shape[-1]
    h1 = h1_ref[...]
    aggx = p_ref[0] + p_ref[1]
    aggh1 = q_ref[0] + q_ref[1]
    h2 = _mlp(x + aggx, h1 + aggh1, w1T_ref[...], b1_ref[...], g_ref[...],
              be_ref[...], w2T_ref[...], b2_ref[...], w3T_ref[...], b3_ref[...])
    outWT = outWT_ref[...]
    o_ref[...] = op_ref[...] + jnp.dot(h2, outWT[2 * d:],
                                       preferred_element_type=jnp.float32)


def _full(shape):
    return pl.BlockSpec(shape, lambda i: (0,) * len(shape))


def _ceil_to(v, m):
    return -(-v // m) * m


def kernel(x, edge_index, params):
    n, d = x.shape
    e = edge_index.shape[1]
    nw = _NC * _NS

    cw = _ceil_to(-(-e // (_CHUNK * nw)), 2 * _SCH)
    n_rows_pad = _ceil_to(n + 1, _NS * 8)
    sce = _CHUNK * _SCH

    pad_i = np.arange(sce, dtype=np.int32)
    psrc = jnp.asarray((pad_i % n).reshape(_SCH, _CHUNK))
    pdst = jnp.asarray((n + pad_i % (n_rows_pad - n)).reshape(_SCH, _CHUNK))

    if e % sce == 0:
        idx2 = edge_index.reshape(2 * e // _CHUNK, _CHUNK)
        real_chunks = e // _CHUNK
    else:
        e_pad = _ceil_to(e, sce)
        fill = jnp.arange(e_pad - e, dtype=jnp.int32)
        idx2 = jnp.concatenate(
            [edge_index[0], fill % n,
             edge_index[1], n + fill % (n_rows_pad - n)]).reshape(
                 2 * e_pad // _CHUNK, _CHUNK)
        real_chunks = e_pad // _CHUNK
    dst_base = real_chunks

    p, degp = _make_sc_agg(n_rows_pad, cw, real_chunks, dst_base, d, True)(
        x, idx2, psrc, pdst)

    pr = params
    c0, c1 = pr["convs"]
    inWT = pr["in_W"].T
    inb = pr["in_b"].reshape(1, -1)
    hid = inWT.shape[1]

    def wset(c):
        return (c["W1"].T, c["b1"].reshape(1, -1), c["g"].reshape(1, -1),
                c["be"].reshape(1, -1), c["W2"].T, c["b2"].reshape(1, -1),
                c["W3"].T, c["b3"].reshape(1, -1))

    w1 = wset(c0)
    w2 = wset(c1)
    outWT = pr["out_W"].T
    outb = pr["out_b"].reshape(1, -1)

    rb = 2048
    grid = -(-n // rb)
    mid = w1[0].shape[1]
    h1d = w1[6].shape[1]

    row = lambda s: pl.BlockSpec(s, lambda i: (i, 0))
    row3 = lambda s: pl.BlockSpec(s, lambda i: (0, i, 0))

    h0 = pl.pallas_call(
        _tc_h0_body,
        grid=(grid,),
        in_specs=[row((rb, d)), _full((d, hid)), _full((1, hid))],
        out_specs=[row((rb, hid))],
        out_shape=[jax.ShapeDtypeStruct((n, hid), jnp.float32)],
    )(x, inWT, inb)[0]

    h1 = pl.pallas_call(
        _tc_layer1_body,
        grid=(grid,),
        in_specs=[
            row((rb, d)), row3((_NC, rb, d)),
            pl.BlockSpec((_NC, _NS, rb), lambda i: (0, 0, i)),
            _full((d, hid)), _full((1, hid)),
            _full((d + hid, mid)), _full((1, mid)), _full((1, mid)),
            _full((1, mid)), _full((mid, mid)), _full((1, mid)),
            _full((mid, h1d)), _full((1, h1d)),
        ],
        out_specs=[row((rb, h1d))],
        out_shape=[jax.ShapeDtypeStruct((n, h1d), jnp.float32)],
    )(x, p, degp, inWT, inb, *w1)[0]

    q = _make_sc_agg(n_rows_pad, cw, real_chunks, dst_base, h1d, False)(
        h1, idx2, psrc, pdst)

    out_part = pl.pallas_call(
        _tc_outpart_body,
        grid=(grid,),
        in_specs=[row((rb, hid)), row((rb, h1d)),
                  _full((outWT.shape[0], outWT.shape[1])),
                  _full((1, outWT.shape[1]))],
        out_specs=[row((rb, outWT.shape[1]))],
        out_shape=[jax.ShapeDtypeStruct((n, outWT.shape[1]), jnp.float32)],
    )(h0, h1, outWT, outb)[0]

    mid2 = w2[0].shape[1]
    out = pl.pallas_call(
        _tc_layer2_body,
        grid=(grid,),
        in_specs=[
            row((rb, d)), row((rb, h1d)),
            row3((_NC, rb, d)), row3((_NC, rb, h1d)),
            row((rb, outWT.shape[1])),
            _full((d + h1d, mid2)), _full((1, mid2)), _full((1, mid2)),
            _full((1, mid2)), _full((mid2, mid2)), _full((1, mid2)),
            _full((mid2, w2[6].shape[1])), _full((1, w2[6].shape[1])),
            _full((outWT.shape[0], outWT.shape[1])),
        ],
        out_specs=[row((rb, outWT.shape[1]))],
        out_shape=[jax.ShapeDtypeStruct((n, outWT.shape[1]), jnp.float32)],
    )(x, h1, p, q, out_part, *w2, outWT)[0]

    return out


# speedup vs baseline: 1.0319x; 1.0319x over previous
import jax
import jax.numpy as jnp
import numpy as np
from jax import lax
from jax.experimental import pallas as pl
from jax.experimental.pallas import tpu as pltpu
from jax.experimental.pallas import tpu_sc as plsc

_NC = 2
_NS = 16
_CHUNK = 64
_SCH = 8
_NBUF = 4
_LANES = 16


def _make_sc_agg(n_rows_pad, cw, real_chunks, dst_base, feat, with_deg):
    mesh = plsc.VectorSubcoreMesh(core_axis_name="c", subcore_axis_name="s")
    outs = [jax.ShapeDtypeStruct((_NC, n_rows_pad, feat), jnp.float32)]
    if with_deg:
        outs.append(jax.ShapeDtypeStruct((_NC, _NS, n_rows_pad), jnp.float32))
    scratch = (
        [pltpu.VMEM((_SCH, _CHUNK), jnp.int32) for _ in range(4)]
        + [pltpu.VMEM((_CHUNK, feat), jnp.float32) for _ in range(_NBUF)]
        + [pltpu.VMEM_SHARED((n_rows_pad, feat), jnp.float32)]
        + [pltpu.SemaphoreType.DMA for _ in range(2 * _NBUF + 2)]
    )
    if with_deg:
        scratch.append(pltpu.VMEM((n_rows_pad,), jnp.float32))

    nsc = cw // _SCH
    stripe = n_rows_pad // _NS
    blocks = []
    r = 0
    while r < stripe:
        b = min(_CHUNK, stripe - r)
        blocks.append((r, b))
        r += b

    def body(tab_hbm, idx_hbm, psrc_hbm, pdst_hbm, *refs):
        if with_deg:
            agg_out, deg_out = refs[0], refs[1]
            rs = refs[2:]
            deg_v = rs[4 + _NBUF + 1 + 2 * _NBUF + 2]
        else:
            agg_out = refs[0]
            deg_out = deg_v = None
            rs = refs[1:]
        srcs = [rs[0], rs[1]]
        dsts = [rs[2], rs[3]]
        bufs = list(rs[4:4 + _NBUF])
        agg_s = rs[4 + _NBUF]
        gsems = list(rs[5 + _NBUF:5 + 2 * _NBUF])
        ssems = list(rs[5 + 2 * _NBUF:5 + 3 * _NBUF])
        isems = list(rs[5 + 3 * _NBUF:7 + 3 * _NBUF])

        cid = lax.axis_index("c")
        sid = lax.axis_index("s")
        wid = sid * _NC + cid
        base = sid * stripe

        zv = jnp.zeros((_LANES,), jnp.float32)
        ov = jnp.ones((_LANES,), jnp.float32)

        def _zero_row(i, c):
            for k2 in range(feat // _LANES):
                bufs[0][i, pl.ds(k2 * _LANES, _LANES)] = zv
            return c

        lax.fori_loop(0, _CHUNK, _zero_row, 0)
        for off, b in blocks:
            pltpu.sync_copy(bufs[0].at[pl.ds(0, b)],
                            agg_s.at[pl.ds(base + off, b)])
        if with_deg:
            def _zero_deg(i, c):
                deg_v[pl.ds(i * _LANES, _LANES)] = zv
                return c

            lax.fori_loop(0, n_rows_pad // _LANES, _zero_deg, 0)

        plsc.subcore_barrier()

        off0 = wid * cw

        @pl.when(off0 < real_chunks)
        def _():
            pltpu.sync_copy(idx_hbm.at[pl.ds(off0, _SCH)], srcs[0])
            pltpu.sync_copy(idx_hbm.at[pl.ds(dst_base + off0, _SCH)],
                            dsts[0])

        @pl.when(off0 >= real_chunks)
        def _():
            pltpu.sync_copy(psrc_hbm, srcs[0])
            pltpu.sync_copy(pdst_hbm, dsts[0])
        pltpu.async_copy(tab_hbm.at[srcs[0].at[0]], bufs[0], gsems[0])
        pltpu.async_copy(tab_hbm.at[srcs[0].at[1]], bufs[1], gsems[1])

        def _outer(i, c):
            for s2 in range(2):
                sc_i = 2 * i + s2
                tb = sc_i * _SCH
                sv, dv = srcs[s2], dsts[s2]
                nsv, ndv = srcs[1 - s2], dsts[1 - s2]
                for j in range(_SCH):
                    t = tb + j
                    b = j % _NBUF
                    b2 = (j + 2) % _NBUF
                    pltpu.make_async_copy(
                        tab_hbm.at[sv.at[0]], bufs[b], gsems[b]).wait()
                    pltpu.async_copy(bufs[b], agg_s.at[dv.at[j]],
                                     ssems[b], add=True)
                    if with_deg:
                        for k2 in range(_CHUNK // _LANES):
                            idx16 = dv[j, pl.ds(k2 * _LANES, _LANES)]
                            plsc.addupdate_scatter(deg_v, [idx16], ov)

                    @pl.when(t + 2 < cw)
                    def _():
                        @pl.when(t >= 2)
                        def _():
                            pltpu.make_async_copy(
                                bufs[b2], agg_s.at[dv.at[0]],
                                ssems[b2]).wait()
                        if j < _SCH - 2:
                            pltpu.async_copy(tab_hbm.at[sv.at[j + 2]],
                                             bufs[b2], gsems[b2])
                        else:
                            if j == _SCH - 2:
                                pltpu.make_async_copy(
                                    psrc_hbm, nsv, isems[1 - s2]).wait()
                                pltpu.make_async_copy(
                                    pdst_hbm, ndv, isems[1 - s2]).wait()
                            pltpu.async_copy(
                                tab_hbm.at[nsv.at[j + 2 - _SCH]],
                                bufs[b2], gsems[b2])

                    if j == 1:
                        @pl.when(sc_i + 1 < nsc)
                        def _():
                            offc = wid * cw + (sc_i + 1) * _SCH

                            @pl.when(offc < real_chunks)
                            def _():
                                pltpu.async_copy(
                                    idx_hbm.at[pl.ds(offc, _SCH)],
                                    nsv, isems[1 - s2])
                                pltpu.async_copy(
                                    idx_hbm.at[pl.ds(dst_base + offc, _SCH)],
                                    ndv, isems[1 - s2])

                            @pl.when(offc >= real_chunks)
                            def _():
                                pltpu.async_copy(psrc_hbm, nsv,
                                                 isems[1 - s2])
                                pltpu.async_copy(pdst_hbm, ndv,
                                                 isems[1 - s2])
            return c

        lax.fori_loop(0, nsc // 2, _outer, 0)

        for b in range(_NBUF):
            pltpu.make_async_copy(bufs[b], agg_s.at[dsts[0].at[0]],
                                  ssems[b]).wait()

        plsc.subcore_barrier()

        pltpu.sync_copy(agg_s.at[pl.ds(base, stripe)],
                        agg_out.at[cid, pl.ds(base, stripe)])
        if with_deg:
            pltpu.sync_copy(deg_v, deg_out.at[cid, sid])

    return pl.kernel(body, out_type=tuple(outs) if with_deg else outs[0],
                     mesh=mesh, scratch_types=scratch,
                     compiler_params=pltpu.CompilerParams(
                         needs_layout_passes=False))


def _lrelu(t):
    return jnp.where(t > 0, t, 0.1 * t)


def _mlp(za, zb, w1T, b1, g, be, w2T, b2, w3T, b3):
    d = za.shape[-1]
    t = (jnp.dot(za, w1T[:d], preferred_element_type=jnp.float32)
         + jnp.dot(zb, w1T[d:], preferred_element_type=jnp.float32) + b1)
    mu = jnp.mean(t, axis=-1, keepdims=True)
    var = jnp.mean((t - mu) ** 2, axis=-1, keepdims=True)
    t = (t - mu) / jnp.sqrt(var + 1e-5) * g + be
    t = _lrelu(t)
    u = _lrelu(jnp.dot(t, w2T, preferred_element_type=jnp.float32) + b2)
    return jnp.dot(u, w3T, preferred_element_type=jnp.float32) + b3


def _tc_layer1_body(x_ref, p_ref, degp_ref, inWT_ref, inb_ref,
                    w1T_ref, b1_ref, g_ref, be_ref, w2T_ref, b2_ref,
                    w3T_ref, b3_ref, h0_ref, h1_ref):
    x = x_ref[...]
    d = x.shape[-1]
    aggx = p_ref[0] + p_ref[1]
    deg = jnp.sum(degp_ref[...], axis=(0, 1))[:, None]
    inWT = inWT_ref[...]
    inb = inb_ref[...]
    h0 = jnp.dot(x, inWT, preferred_element_type=jnp.float32) + inb
    aggh0 = jnp.dot(aggx, inWT, preferred_element_type=jnp.float32) + deg * inb
    h1 = _mlp(x + aggx, h0 + aggh0, w1T_ref[...], b1_ref[...], g_ref[...],
              be_ref[...], w2T_ref[...], b2_ref[...], w3T_ref[...], b3_ref[...])
    h0_ref[...] = h0
    h1_ref[...] = h1


def _tc_layer2_body(x_ref, h0_ref, h1_ref, p_ref, q_ref,
                    w1T_ref, b1_ref, g_ref, be_ref, w2T_ref, b2_ref,
                    w3T_ref, b3_ref, outWT_ref, outb_ref, o_ref):
    x = x_ref[...]
    d = x.shape[-1]
    h0 = h0_ref[...]
    h1 = h1_ref[...]
    aggx = p_ref[0] + p_ref[1]
    aggh1 = q_ref[0] + q_ref[1]
    h2 = _mlp(x + aggx, h1 + aggh1, w1T_ref[...], b1_ref[...], g_ref[...],
              be_ref[...], w2T_ref[...], b2_ref[...], w3T_ref[...], b3_ref[...])
    outWT = outWT_ref[...]
    o_ref[...] = (jnp.dot(h0, outWT[:d], preferred_element_type=jnp.float32)
                  + jnp.dot(h1, outWT[d:2 * d], preferred_element_type=jnp.float32)
                  + jnp.dot(h2, outWT[2 * d:], preferred_element_type=jnp.float32)
                  + outb_ref[...])


def _full(shape):
    return pl.BlockSpec(shape, lambda i: (0,) * len(shape))


def _ceil_to(v, m):
    return -(-v // m) * m


def kernel(x, edge_index, params):
    n, d = x.shape
    e = edge_index.shape[1]
    nw = _NC * _NS

    cw = _ceil_to(-(-e // (_CHUNK * nw)), 2 * _SCH)
    n_rows_pad = _ceil_to(n + 1, _NS * 8)
    sce = _CHUNK * _SCH

    pad_i = np.arange(sce, dtype=np.int32)
    psrc = jnp.asarray((pad_i % n).reshape(_SCH, _CHUNK))
    pdst = jnp.asarray((n + pad_i % (n_rows_pad - n)).reshape(_SCH, _CHUNK))

    if e % sce == 0:
        idx2 = edge_index.reshape(2 * e // _CHUNK, _CHUNK)
        real_chunks = e // _CHUNK
    else:
        e_pad = _ceil_to(e, sce)
        fill = jnp.arange(e_pad - e, dtype=jnp.int32)
        idx2 = jnp.concatenate(
            [edge_index[0], fill % n,
             edge_index[1], n + fill % (n_rows_pad - n)]).reshape(
                 2 * e_pad // _CHUNK, _CHUNK)
        real_chunks = e_pad // _CHUNK
    dst_base = real_chunks

    p, degp = _make_sc_agg(n_rows_pad, cw, real_chunks, dst_base, d, True)(
        x, idx2, psrc, pdst)

    pr = params
    c0, c1 = pr["convs"]
    inWT = pr["in_W"].T
    inb = pr["in_b"].reshape(1, -1)
    hid = inWT.shape[1]

    def wset(c):
        return (c["W1"].T, c["b1"].reshape(1, -1), c["g"].reshape(1, -1),
                c["be"].reshape(1, -1), c["W2"].T, c["b2"].reshape(1, -1),
                c["W3"].T, c["b3"].reshape(1, -1))

    w1 = wset(c0)
    w2 = wset(c1)
    outWT = pr["out_W"].T
    outb = pr["out_b"].reshape(1, -1)

    rb = 2048
    grid = -(-n // rb)
    mid = w1[0].shape[1]
    h1d = w1[6].shape[1]

    row = lambda s: pl.BlockSpec(s, lambda i: (i, 0))
    row3 = lambda s: pl.BlockSpec(s, lambda i: (0, i, 0))

    h0, h1 = pl.pallas_call(
        _tc_layer1_body,
        grid=(grid,),
        in_specs=[
            row((rb, d)), row3((_NC, rb, d)),
            pl.BlockSpec((_NC, _NS, rb), lambda i: (0, 0, i)),
            _full((d, hid)), _full((1, hid)),
            _full((d + hid, mid)), _full((1, mid)), _full((1, mid)),
            _full((1, mid)), _full((mid, mid)), _full((1, mid)),
            _full((mid, h1d)), _full((1, h1d)),
        ],
        out_specs=[row((rb, hid)), row((rb, h1d))],
        out_shape=[jax.ShapeDtypeStruct((n, hid), jnp.float32),
                   jax.ShapeDtypeStruct((n, h1d), jnp.float32)],
    )(x, p, degp, inWT, inb, *w1)

    q = _make_sc_agg(n_rows_pad, cw, real_chunks, dst_base, h1d, False)(
        h1, idx2, psrc, pdst)

    mid2 = w2[0].shape[1]
    out = pl.pallas_call(
        _tc_layer2_body,
        grid=(grid,),
        in_specs=[
            row((rb, d)), row((rb, hid)), row((rb, h1d)),
            row3((_NC, rb, d)), row3((_NC, rb, h1d)),
            _full((d + h1d, mid2)), _full((1, mid2)), _full((1, mid2)),
            _full((1, mid2)), _full((mid2, mid2)), _full((1, mid2)),
            _full((mid2, w2[6].shape[1])), _full((1, w2[6].shape[1])),
            _full((outWT.shape[0], outWT.shape[1])),
            _full((1, outWT.shape[1])),
        ],
        out_specs=[row((rb, outWT.shape[1]))],
        out_shape=[jax.ShapeDtypeStruct((n, outWT.shape[1]), jnp.float32)],
    )(x, h0, h1, p, q, *w2, outWT, outb)[0]

    return out
